# Initial kernel scaffold; baseline (speedup 1.0000x reference)
import jax
import jax.numpy as jnp
from jax.experimental import pallas as pl


def kernel(user_emb, item_emb, adj_vals, mm_vals, adj_row, adj_col, mm_row, mm_col):
    raise NotImplementedError("write your pallas kernel here")


# speedup vs baseline: 7.3466x; 7.3466x over previous
import functools

import jax
import jax.numpy as jnp
from jax import lax
from jax.experimental import pallas as pl
from jax.experimental.pallas import tpu as pltpu
from jax.experimental.pallas import tpu_sc as plsc

NC = 2
NS = 16
L = 16
DH = 32
CHUNK = 128
BLK = 6
BLK_E = CHUNK * BLK

_SPLAT_DNUMS = lax.GatherDimensionNumbers(
    offset_dims=(), collapsed_slice_dims=(0,), start_index_map=(0,))


def _spmm_sc(n_dst, n_src, n_blocks):
  kps = n_blocks * BLK
  nfull = n_dst // BLK_E
  tail = n_dst - nfull * BLK_E

  mesh = plsc.VectorSubcoreMesh(
      core_axis_name="c", subcore_axis_name="s", num_cores=NC,
      num_subcores=NS)

  @functools.partial(
      pl.kernel,
      out_type=jax.ShapeDtypeStruct((NC * n_dst, DH), jnp.float32),
      mesh=mesh,
      scratch_types=[
          pltpu.VMEM((BLK, CHUNK), jnp.int32),
          pltpu.VMEM((BLK, CHUNK), jnp.int32),
          pltpu.VMEM((BLK, CHUNK), jnp.float32),
          pltpu.VMEM((BLK_E, DH), jnp.float32),
          pltpu.VMEM_SHARED((n_dst, DH), jnp.float32),
          pltpu.SemaphoreType.DMA,
      ],
      compiler_params=pltpu.CompilerParams(use_tc_tiling_on_sc=False),
  )
  def spmm(col_hbm, row_hbm, val_hbm, x_hbm, y_hbm,
           colv, rowv, valv, rows, acc, sem):
    c = lax.axis_index("c")
    s = lax.axis_index("s")
    zeros16 = jnp.zeros((L,), jnp.float32)

    def zrow(k, _):
      rows[k, pl.ds(0, L)] = zeros16
      rows[k, pl.ds(L, L)] = zeros16
      return 0
    lax.fori_loop(0, BLK_E, zrow, 0)
    zcnt = (nfull - s + NS - 1) // NS

    def zchunk(i, _):
      k = s + i * NS
      pltpu.sync_copy(rows, acc.at[pl.ds(k * BLK_E, BLK_E)])
      return 0
    lax.fori_loop(0, zcnt, zchunk, 0)
    if tail:
      @pl.when(s == 0)
      def _():
        pltpu.sync_copy(rows.at[pl.ds(0, tail)],
                        acc.at[pl.ds(nfull * BLK_E, tail)])
    plsc.subcore_barrier()

    col_off = c * n_src

    def block_body(b, _):
      base = s * kps + b * BLK
      pltpu.sync_copy(col_hbm.at[pl.ds(base, BLK)], colv)
      pltpu.sync_copy(row_hbm.at[pl.ds(base, BLK)], rowv)
      pltpu.sync_copy(val_hbm.at[pl.ds(base, BLK)], valv)

      def off_body(t, _):
        j = t // (CHUNK // L)
        g = t % (CHUNK // L)
        colv[j, pl.ds(g * L, L)] = colv[j, pl.ds(g * L, L)] + col_off
        return 0
      lax.fori_loop(0, BLK * (CHUNK // L), off_body, 0)

      cps = []
      for j in range(BLK):
        cps.append(pltpu.async_copy(
            x_hbm.at[colv.at[j]], rows.at[pl.ds(j * CHUNK, CHUNK)], sem))
      for cp in cps:
        cp.wait()

      def scale_body(t, _):
        j = t // (CHUNK // L)
        g = t % (CHUNK // L)
        vv = valv[j, pl.ds(g * L, L)]
        k0 = j * CHUNK + g * L
        for e in range(L):
          sp = lax.gather(
              vv, jnp.full((L, 1), e, jnp.int32), _SPLAT_DNUMS, (1,),
              mode=lax.GatherScatterMode.PROMISE_IN_BOUNDS)
          rows[k0 + e, pl.ds(0, L)] = rows[k0 + e, pl.ds(0, L)] * sp
          rows[k0 + e, pl.ds(L, L)] = rows[k0 + e, pl.ds(L, L)] * sp
        return 0
      lax.fori_loop(0, BLK * (CHUNK // L), scale_body, 0)

      for j in range(BLK):
        pltpu.sync_copy(rows.at[pl.ds(j * CHUNK, CHUNK)],
                        acc.at[rowv.at[j]], add=True)
      return 0
    lax.fori_loop(0, n_blocks, block_body, 0)
    plsc.subcore_barrier()

    ybase = c * n_dst

    def wchunk(i, _):
      k = s + i * NS
      pltpu.sync_copy(acc.at[pl.ds(k * BLK_E, BLK_E)], rows)
      pltpu.sync_copy(rows, y_hbm.at[pl.ds(ybase + k * BLK_E, BLK_E)])
      return 0
    lax.fori_loop(0, zcnt, wchunk, 0)
    if tail:
      @pl.when(s == 0)
      def _():
        pltpu.sync_copy(acc.at[pl.ds(nfull * BLK_E, tail)],
                        rows.at[pl.ds(0, tail)])
        pltpu.sync_copy(rows.at[pl.ds(0, tail)],
                        y_hbm.at[pl.ds(ybase + nfull * BLK_E, tail)])

  return spmm


def _pad_edges(row, col, val, n_blocks, n_dst, n_src):
  e_pad = n_blocks * NS * BLK_E
  pad = e_pad - row.shape[0]
  spread = jnp.arange(pad, dtype=jnp.int32)
  row = jnp.concatenate([row.astype(jnp.int32), spread % n_dst])
  col = jnp.concatenate([col.astype(jnp.int32), spread % n_src])
  val = jnp.concatenate([val, jnp.zeros((pad,), jnp.float32)])
  k = e_pad // CHUNK
  return (row.reshape(k, CHUNK), col.reshape(k, CHUNK),
          val.reshape(k, CHUNK))


def _combine_u(y0, y1, y2, n_users, n_nodes):
  bs = 2000

  def body(a0, a1, a2, b0, b1, b2, out):
    out[:, :DH] = (a0[...] + a1[...] + a2[...]) * (1.0 / 3.0)
    out[:, DH:] = (b0[...] + b1[...] + b2[...]) * (1.0 / 3.0)

  lo = pl.BlockSpec((bs, DH), lambda i: (i, 0))
  hi = pl.BlockSpec((bs, DH), lambda i: (i + n_nodes // bs, 0))
  return pl.pallas_call(
      body,
      grid=(n_users // bs,),
      in_specs=[lo, lo, lo, hi, hi, hi],
      out_specs=pl.BlockSpec((bs, 2 * DH), lambda i: (i, 0)),
      out_shape=jax.ShapeDtypeStruct((n_users, 2 * DH), jnp.float32),
  )(y0, y1, y2, y0, y1, y2)


def _combine_i(y0, y1, y2, h, n_users, n_items, n_nodes):
  bs = 2000

  def body(a0, a1, a2, ha, b0, b1, b2, hb, out):
    out[:, :DH] = (a0[...] + a1[...] + a2[...]) * (1.0 / 3.0) + ha[...]
    out[:, DH:] = (b0[...] + b1[...] + b2[...]) * (1.0 / 3.0) + hb[...]

  lo = pl.BlockSpec((bs, DH), lambda i: (i + n_users // bs, 0))
  hi = pl.BlockSpec((bs, DH), lambda i: (i + (n_nodes + n_users) // bs, 0))
  hlo = pl.BlockSpec((bs, DH), lambda i: (i, 0))
  hhi = pl.BlockSpec((bs, DH), lambda i: (i + n_items // bs, 0))
  return pl.pallas_call(
      body,
      grid=(n_items // bs,),
      in_specs=[lo, lo, lo, hlo, hi, hi, hi, hhi],
      out_specs=pl.BlockSpec((bs, 2 * DH), lambda i: (i, 0)),
      out_shape=jax.ShapeDtypeStruct((n_items, 2 * DH), jnp.float32),
  )(y0, y1, y2, h, y0, y1, y2, h)


def kernel(user_emb, item_emb, adj_vals, mm_vals, adj_row, adj_col,
           mm_row, mm_col):
  n_users, n_items = user_emb.shape[0], item_emb.shape[0]
  n_nodes = n_users + n_items

  ego0 = jnp.concatenate([user_emb, item_emb], axis=0)
  x0 = jnp.concatenate([ego0[:, :DH], ego0[:, DH:]], axis=0)
  it = jnp.concatenate([item_emb[:, :DH], item_emb[:, DH:]], axis=0)

  per_sub = NS * BLK_E
  ui_blocks = -(-adj_row.shape[0] // per_sub)
  mm_blocks = -(-mm_row.shape[0] // per_sub)
  arow, acol, aval = _pad_edges(adj_row, adj_col, adj_vals, ui_blocks,
                                n_nodes, n_nodes)
  mrow, mcol, mval = _pad_edges(mm_row, mm_col, mm_vals, mm_blocks,
                                n_items, n_items)

  spmm_ui = _spmm_sc(n_nodes, n_nodes, ui_blocks)
  spmm_mm = _spmm_sc(n_items, n_items, mm_blocks)

  h = spmm_mm(mcol, mrow, mval, it)
  y1 = spmm_ui(acol, arow, aval, x0)
  y2 = spmm_ui(acol, arow, aval, y1)

  u = _combine_u(x0, y1, y2, n_users, n_nodes)
  i = _combine_i(x0, y1, y2, h, n_users, n_items, n_nodes)
  return (u, i)


# speedup vs baseline: 10.9653x; 1.4926x over previous
import functools

import jax
import jax.numpy as jnp
from jax import lax
from jax.experimental import pallas as pl
from jax.experimental.pallas import tpu as pltpu
from jax.experimental.pallas import tpu_sc as plsc

NC = 2
NS = 16
L = 16
DH = 32
CHUNK = 128
BLK = 2
BLK_E = CHUNK * BLK
SB = 16
SB_C = SB * BLK

_SPLAT_DNUMS = lax.GatherDimensionNumbers(
    offset_dims=(), collapsed_slice_dims=(0,), start_index_map=(0,))


def _spmm_sc(n_dst, n_src, n_blocks):
  kps = n_blocks * BLK
  n_sb = n_blocks // SB
  sb_tail = n_blocks - n_sb * SB
  nfull = n_dst // BLK_E
  tail = n_dst - nfull * BLK_E

  mesh = plsc.VectorSubcoreMesh(
      core_axis_name="c", subcore_axis_name="s", num_cores=NC,
      num_subcores=NS)

  @functools.partial(
      pl.kernel,
      out_type=jax.ShapeDtypeStruct((NC * n_dst, DH), jnp.float32),
      mesh=mesh,
      scratch_types=[
          pltpu.VMEM((SB_C, CHUNK), jnp.int32),
          pltpu.VMEM((SB_C, CHUNK), jnp.int32),
          pltpu.VMEM((SB_C, CHUNK), jnp.float32),
          pltpu.VMEM((2, BLK_E, DH), jnp.float32),
          pltpu.VMEM_SHARED((n_dst, DH), jnp.float32),
          pltpu.SemaphoreType.DMA,
          pltpu.SemaphoreType.DMA,
          pltpu.SemaphoreType.DMA,
          pltpu.SemaphoreType.DMA,
      ],
      compiler_params=pltpu.CompilerParams(use_tc_tiling_on_sc=False),
  )
  def spmm(col_hbm, row_hbm, val_hbm, x_hbm, y_hbm,
           colv, rowv, valv, rows2, acc, semg0, semg1, sems0, sems1):
    c = lax.axis_index("c")
    s = lax.axis_index("s")
    zeros16 = jnp.zeros((L,), jnp.float32)
    semg = (semg0, semg1)
    sems = (sems0, sems1)

    def zrow(k, _):
      rows2[0, k, pl.ds(0, L)] = zeros16
      rows2[0, k, pl.ds(L, L)] = zeros16
      return 0
    lax.fori_loop(0, BLK_E, zrow, 0)
    zcnt = (nfull - s + NS - 1) // NS

    def zchunk(i, _):
      k = s + i * NS
      pltpu.sync_copy(rows2.at[0], acc.at[pl.ds(k * BLK_E, BLK_E)])
      return 0
    lax.fori_loop(0, zcnt, zchunk, 0)
    if tail:
      @pl.when(s == 0)
      def _():
        pltpu.sync_copy(rows2.at[0, pl.ds(0, tail)],
                        acc.at[pl.ds(nfull * BLK_E, tail)])
    plsc.subcore_barrier()

    col_off = c * n_src

    def stage(sb):
      base = s * kps + sb * SB_C
      nch = SB_C if not isinstance(sb, int) or sb < n_sb else 2 * sb_tail
      pltpu.sync_copy(col_hbm.at[pl.ds(base, nch)], colv.at[pl.ds(0, nch)])
      pltpu.sync_copy(row_hbm.at[pl.ds(base, nch)], rowv.at[pl.ds(0, nch)])
      pltpu.sync_copy(val_hbm.at[pl.ds(base, nch)], valv.at[pl.ds(0, nch)])

    def drain_scatter(p, n=BLK):
      for _ in range(n):
        pltpu.make_async_copy(rows2.at[p, pl.ds(0, CHUNK)],
                              acc.at[rowv.at[0]], sems[p]).wait()

    def fire(b, p):
      ch0 = b * BLK
      for t in range(BLK * (CHUNK // L)):
        j, g = ch0 + t // (CHUNK // L), t % (CHUNK // L)
        colv[j, pl.ds(g * L, L)] = colv[j, pl.ds(g * L, L)] + col_off
      for j in range(BLK):
        pltpu.async_copy(x_hbm.at[colv.at[ch0 + j]],
                         rows2.at[p, pl.ds(j * CHUNK, CHUNK)], semg[p])

    def process(b, p):
      for _ in range(BLK):
        pltpu.make_async_copy(x_hbm.at[colv.at[0]],
                              rows2.at[p, pl.ds(0, CHUNK)], semg[p]).wait()
      ch0 = b * BLK

      def scale_body(t, _):
        j = t // (CHUNK // L)
        g = t % (CHUNK // L)
        vv = valv[ch0 + j, pl.ds(g * L, L)]
        k0 = j * CHUNK + g * L
        for e in range(L):
          sp = lax.gather(
              vv, jnp.full((L, 1), e, jnp.int32), _SPLAT_DNUMS, (1,),
              mode=lax.GatherScatterMode.PROMISE_IN_BOUNDS)
          rows2[p, k0 + e, pl.ds(0, L)] = rows2[p, k0 + e, pl.ds(0, L)] * sp
          rows2[p, k0 + e, pl.ds(L, L)] = rows2[p, k0 + e, pl.ds(L, L)] * sp
        return 0
      lax.fori_loop(0, BLK * (CHUNK // L), scale_body, 0)
      for j in range(BLK):
        pltpu.async_copy(rows2.at[p, pl.ds(j * CHUNK, CHUNK)],
                         acc.at[rowv.at[ch0 + j]], sems[p], add=True)

    def sb_body(sb, _):
      @pl.when(sb > 0)
      def _():
        process(SB - 1, 1)
        drain_scatter(0)
        drain_scatter(1)
      stage(sb)

      def pair_body(jp, _):
        b0 = 2 * jp

        @pl.when(jp > 0)
        def _():
          drain_scatter(0)
        fire(b0, 0)

        @pl.when(jp > 0)
        def _():
          process(b0 - 1, 1)

        @pl.when(jp > 0)
        def _():
          drain_scatter(1)
        fire(b0 + 1, 1)
        process(b0, 0)
        return 0
      lax.fori_loop(0, SB // 2, pair_body, 0)
      return 0
    lax.fori_loop(0, n_sb, sb_body, 0)

    if sb_tail:
      if n_sb:
        process(SB - 1, 1)
        drain_scatter(0)
        drain_scatter(1)
      stage(n_sb)
      for b in range(sb_tail):
        p = b % 2
        if b >= 2:
          drain_scatter(p)
        fire(b, p)
        if b >= 1:
          process(b - 1, 1 - p)
      process(sb_tail - 1, (sb_tail - 1) % 2)
      drain_scatter(0, n=BLK if sb_tail > 1 else 0)
      drain_scatter(1 if sb_tail > 1 else 0, n=BLK)
    elif n_sb:
      process(SB - 1, 1)
      drain_scatter(0)
      drain_scatter(1)
    plsc.subcore_barrier()

    ybase = c * n_dst

    def wchunk(i, _):
      k = s + i * NS
      pltpu.sync_copy(acc.at[pl.ds(k * BLK_E, BLK_E)], rows2.at[0])
      pltpu.sync_copy(rows2.at[0], y_hbm.at[pl.ds(ybase + k * BLK_E, BLK_E)])
      return 0
    lax.fori_loop(0, zcnt, wchunk, 0)
    if tail:
      @pl.when(s == 0)
      def _():
        pltpu.sync_copy(acc.at[pl.ds(nfull * BLK_E, tail)],
                        rows2.at[0, pl.ds(0, tail)])
        pltpu.sync_copy(rows2.at[0, pl.ds(0, tail)],
                        y_hbm.at[pl.ds(ybase + nfull * BLK_E, tail)])

  return spmm


def _pad_edges(row, col, val, n_blocks, n_dst, n_src):
  e_pad = n_blocks * NS * BLK_E
  pad = e_pad - row.shape[0]
  spread = jnp.arange(pad, dtype=jnp.int32)
  row = jnp.concatenate([row.astype(jnp.int32), spread % n_dst])
  col = jnp.concatenate([col.astype(jnp.int32), spread % n_src])
  val = jnp.concatenate([val, jnp.zeros((pad,), jnp.float32)])
  k = e_pad // CHUNK
  return (row.reshape(k, CHUNK), col.reshape(k, CHUNK),
          val.reshape(k, CHUNK))


def _combine_u(y0, y1, y2, n_users, n_nodes):
  bs = 2000

  def body(a0, a1, a2, b0, b1, b2, out):
    out[:, :DH] = (a0[...] + a1[...] + a2[...]) * (1.0 / 3.0)
    out[:, DH:] = (b0[...] + b1[...] + b2[...]) * (1.0 / 3.0)

  lo = pl.BlockSpec((bs, DH), lambda i: (i, 0))
  hi = pl.BlockSpec((bs, DH), lambda i: (i + n_nodes // bs, 0))
  return pl.pallas_call(
      body,
      grid=(n_users // bs,),
      in_specs=[lo, lo, lo, hi, hi, hi],
      out_specs=pl.BlockSpec((bs, 2 * DH), lambda i: (i, 0)),
      out_shape=jax.ShapeDtypeStruct((n_users, 2 * DH), jnp.float32),
  )(y0, y1, y2, y0, y1, y2)


def _combine_i(y0, y1, y2, h, n_users, n_items, n_nodes):
  bs = 2000

  def body(a0, a1, a2, ha, b0, b1, b2, hb, out):
    out[:, :DH] = (a0[...] + a1[...] + a2[...]) * (1.0 / 3.0) + ha[...]
    out[:, DH:] = (b0[...] + b1[...] + b2[...]) * (1.0 / 3.0) + hb[...]

  lo = pl.BlockSpec((bs, DH), lambda i: (i + n_users // bs, 0))
  hi = pl.BlockSpec((bs, DH), lambda i: (i + (n_nodes + n_users) // bs, 0))
  hlo = pl.BlockSpec((bs, DH), lambda i: (i, 0))
  hhi = pl.BlockSpec((bs, DH), lambda i: (i + n_items // bs, 0))
  return pl.pallas_call(
      body,
      grid=(n_items // bs,),
      in_specs=[lo, lo, lo, hlo, hi, hi, hi, hhi],
      out_specs=pl.BlockSpec((bs, 2 * DH), lambda i: (i, 0)),
      out_shape=jax.ShapeDtypeStruct((n_items, 2 * DH), jnp.float32),
  )(y0, y1, y2, h, y0, y1, y2, h)


def kernel(user_emb, item_emb, adj_vals, mm_vals, adj_row, adj_col,
           mm_row, mm_col):
  n_users, n_items = user_emb.shape[0], item_emb.shape[0]
  n_nodes = n_users + n_items

  ego0 = jnp.concatenate([user_emb, item_emb], axis=0)
  x0 = jnp.concatenate([ego0[:, :DH], ego0[:, DH:]], axis=0)
  it = jnp.concatenate([item_emb[:, :DH], item_emb[:, DH:]], axis=0)

  per_sub = NS * BLK_E
  ui_blocks = -(-adj_row.shape[0] // per_sub)
  mm_blocks = -(-mm_row.shape[0] // per_sub)
  arow, acol, aval = _pad_edges(adj_row, adj_col, adj_vals, ui_blocks,
                                n_nodes, n_nodes)
  mrow, mcol, mval = _pad_edges(mm_row, mm_col, mm_vals, mm_blocks,
                                n_items, n_items)

  spmm_ui = _spmm_sc(n_nodes, n_nodes, ui_blocks)
  spmm_mm = _spmm_sc(n_items, n_items, mm_blocks)

  h = spmm_mm(mcol, mrow, mval, it)
  y1 = spmm_ui(acol, arow, aval, x0)
  y2 = spmm_ui(acol, arow, aval, y1)

  u = _combine_u(x0, y1, y2, n_users, n_nodes)
  i = _combine_i(x0, y1, y2, h, n_users, n_items, n_nodes)
  return (u, i)


# speedup vs baseline: 11.1917x; 1.0206x over previous
import functools

import jax
import jax.numpy as jnp
from jax import lax
from jax.experimental import pallas as pl
from jax.experimental.pallas import tpu as pltpu
from jax.experimental.pallas import tpu_sc as plsc

NC = 2
NS = 16
L = 16
DH = 32
CHUNK = 128
SB_C = 21

_SPLAT_DNUMS = lax.GatherDimensionNumbers(
    offset_dims=(), collapsed_slice_dims=(0,), start_index_map=(0,))


def _spmm_sc(n_dst, n_src, n_chunks):
  n_sb = n_chunks // SB_C
  nfull = n_dst // CHUNK
  tail = n_dst - nfull * CHUNK

  mesh = plsc.VectorSubcoreMesh(
      core_axis_name="c", subcore_axis_name="s", num_cores=NC,
      num_subcores=NS)

  @functools.partial(
      pl.kernel,
      out_type=jax.ShapeDtypeStruct((NC * n_dst, DH), jnp.float32),
      mesh=mesh,
      scratch_types=[
          pltpu.VMEM((2, SB_C, CHUNK), jnp.int32),
          pltpu.VMEM((2, SB_C, CHUNK), jnp.int32),
          pltpu.VMEM((2, SB_C, CHUNK), jnp.float32),
          pltpu.VMEM((3, CHUNK, DH), jnp.float32),
          pltpu.VMEM_SHARED((n_dst, DH), jnp.float32),
          pltpu.SemaphoreType.DMA,
          pltpu.SemaphoreType.DMA,
          pltpu.SemaphoreType.DMA,
          pltpu.SemaphoreType.DMA,
          pltpu.SemaphoreType.DMA,
          pltpu.SemaphoreType.DMA,
          pltpu.SemaphoreType.DMA,
      ],
      compiler_params=pltpu.CompilerParams(use_tc_tiling_on_sc=False),
  )
  def spmm(col_hbm, row_hbm, val_hbm, x_hbm, y_hbm,
           colv, rowv, valv, rows3, acc,
           semi, semg0, semg1, semg2, sems0, sems1, sems2):
    c = lax.axis_index("c")
    s = lax.axis_index("s")
    zeros16 = jnp.zeros((L,), jnp.float32)
    semg = (semg0, semg1, semg2)
    sems = (sems0, sems1, sems2)
    col_off = c * n_src

    def zrow(k, _):
      rows3[0, k, pl.ds(0, L)] = zeros16
      rows3[0, k, pl.ds(L, L)] = zeros16
      return 0
    lax.fori_loop(0, CHUNK, zrow, 0)
    zcnt = (nfull - s + NS - 1) // NS

    def zchunk(i, _):
      k = s + i * NS
      pltpu.sync_copy(rows3.at[0], acc.at[pl.ds(k * CHUNK, CHUNK)])
      return 0
    lax.fori_loop(0, zcnt, zchunk, 0)
    if tail:
      @pl.when(s == 0)
      def _():
        pltpu.sync_copy(rows3.at[0, pl.ds(0, tail)],
                        acc.at[pl.ds(nfull * CHUNK, tail)])
    plsc.subcore_barrier()

    def stage(sb, sbp):
      base = s * n_chunks + sb * SB_C
      pltpu.async_copy(col_hbm.at[pl.ds(base, SB_C)], colv.at[sbp], semi)
      pltpu.async_copy(row_hbm.at[pl.ds(base, SB_C)], rowv.at[sbp], semi)
      pltpu.async_copy(val_hbm.at[pl.ds(base, SB_C)], valv.at[sbp], semi)

    def wait_stage(sbp):
      for dst in (colv, rowv, valv):
        pltpu.make_async_copy(col_hbm.at[pl.ds(0, SB_C)], dst.at[0],
                              semi).wait()

    def drain_scatter(p):
      pltpu.make_async_copy(rows3.at[p], acc.at[rowv.at[0, 0]],
                            sems[p]).wait()

    def fire(k, sbp, p):
      for g in range(CHUNK // L):
        colv[sbp, k, pl.ds(g * L, L)] = (
            colv[sbp, k, pl.ds(g * L, L)] + col_off)
      pltpu.async_copy(x_hbm.at[colv.at[sbp, k]], rows3.at[p], semg[p])

    def process(k, sbp, p):
      pltpu.make_async_copy(x_hbm.at[colv.at[0, 0]], rows3.at[p],
                            semg[p]).wait()

      def scale_body(g, _):
        vv = valv[sbp, k, pl.ds(g * L, L)]
        k0 = g * L
        for e in range(L):
          sp = lax.gather(
              vv, jnp.full((L, 1), e, jnp.int32), _SPLAT_DNUMS, (1,),
              mode=lax.GatherScatterMode.PROMISE_IN_BOUNDS)
          rows3[p, k0 + e, pl.ds(0, L)] = rows3[p, k0 + e, pl.ds(0, L)] * sp
          rows3[p, k0 + e, pl.ds(L, L)] = rows3[p, k0 + e, pl.ds(L, L)] * sp
        return 0
      lax.fori_loop(0, CHUNK // L, scale_body, 0)
      pltpu.async_copy(rows3.at[p], acc.at[rowv.at[sbp, k]], sems[p],
                       add=True)

    stage(0, 0)

    def sb_body(sb, _):
      sbp = sb % 2
      wait_stage(sbp)

      def triple(jt, _):
        for i in range(3):
          k = 3 * jt + i
          g = sb * SB_C + k

          @pl.when(g >= 3)
          def _():
            drain_scatter(i)
          fire(k, sbp, i)
          if i == 0:
            @pl.when(jnp.logical_and(jt == 1, sb < n_sb - 1))
            def _():
              stage(sb + 1, 1 - sbp)
          pk = jnp.where(k == 0, SB_C - 1, k - 1)
          psb = jnp.where(k == 0, 1 - sbp, sbp)

          @pl.when(g >= 1)
          def _():
            process(pk, psb, (i + 2) % 3)
        return 0
      lax.fori_loop(0, SB_C // 3, triple, 0)
      return 0
    lax.fori_loop(0, n_sb, sb_body, 0)

    process(SB_C - 1, (n_sb - 1) % 2, 2)
    drain_scatter(0)
    drain_scatter(1)
    drain_scatter(2)
    plsc.subcore_barrier()

    ybase = c * n_dst

    def wchunk(i, _):
      k = s + i * NS
      pltpu.sync_copy(acc.at[pl.ds(k * CHUNK, CHUNK)], rows3.at[0])
      pltpu.sync_copy(rows3.at[0], y_hbm.at[pl.ds(ybase + k * CHUNK, CHUNK)])
      return 0
    lax.fori_loop(0, zcnt, wchunk, 0)
    if tail:
      @pl.when(s == 0)
      def _():
        pltpu.sync_copy(acc.at[pl.ds(nfull * CHUNK, tail)],
                        rows3.at[0, pl.ds(0, tail)])
        pltpu.sync_copy(rows3.at[0, pl.ds(0, tail)],
                        y_hbm.at[pl.ds(ybase + nfull * CHUNK, tail)])

  return spmm


def _pad_edges(row, col, val, n_chunks):
  e_pad = n_chunks * NS * CHUNK
  pad = e_pad - row.shape[0]
  n_dst = 1
  return pad


def _pad_edges2(row, col, val, n_chunks, n_dst, n_src):
  e_pad = n_chunks * NS * CHUNK
  pad = e_pad - row.shape[0]
  spread = jnp.arange(pad, dtype=jnp.int32)
  row = jnp.concatenate([row.astype(jnp.int32), spread % n_dst])
  col = jnp.concatenate([col.astype(jnp.int32), spread % n_src])
  val = jnp.concatenate([val, jnp.zeros((pad,), jnp.float32)])
  k = e_pad // CHUNK
  return (row.reshape(k, CHUNK), col.reshape(k, CHUNK),
          val.reshape(k, CHUNK))


def _combine_u(y0, y1, y2, n_users, n_nodes):
  bs = 2000

  def body(a0, a1, a2, b0, b1, b2, out):
    out[:, :DH] = (a0[...] + a1[...] + a2[...]) * (1.0 / 3.0)
    out[:, DH:] = (b0[...] + b1[...] + b2[...]) * (1.0 / 3.0)

  lo = pl.BlockSpec((bs, DH), lambda i: (i, 0))
  hi = pl.BlockSpec((bs, DH), lambda i: (i + n_nodes // bs, 0))
  return pl.pallas_call(
      body,
      grid=(n_users // bs,),
      in_specs=[lo, lo, lo, hi, hi, hi],
      out_specs=pl.BlockSpec((bs, 2 * DH), lambda i: (i, 0)),
      out_shape=jax.ShapeDtypeStruct((n_users, 2 * DH), jnp.float32),
  )(y0, y1, y2, y0, y1, y2)


def _combine_i(y0, y1, y2, h, n_users, n_items, n_nodes):
  bs = 2000

  def body(a0, a1, a2, ha, b0, b1, b2, hb, out):
    out[:, :DH] = (a0[...] + a1[...] + a2[...]) * (1.0 / 3.0) + ha[...]
    out[:, DH:] = (b0[...] + b1[...] + b2[...]) * (1.0 / 3.0) + hb[...]

  lo = pl.BlockSpec((bs, DH), lambda i: (i + n_users // bs, 0))
  hi = pl.BlockSpec((bs, DH), lambda i: (i + (n_nodes + n_users) // bs, 0))
  hlo = pl.BlockSpec((bs, DH), lambda i: (i, 0))
  hhi = pl.BlockSpec((bs, DH), lambda i: (i + n_items // bs, 0))
  return pl.pallas_call(
      body,
      grid=(n_items // bs,),
      in_specs=[lo, lo, lo, hlo, hi, hi, hi, hhi],
      out_specs=pl.BlockSpec((bs, 2 * DH), lambda i: (i, 0)),
      out_shape=jax.ShapeDtypeStruct((n_items, 2 * DH), jnp.float32),
  )(y0, y1, y2, h, y0, y1, y2, h)


def kernel(user_emb, item_emb, adj_vals, mm_vals, adj_row, adj_col,
           mm_row, mm_col):
  n_users, n_items = user_emb.shape[0], item_emb.shape[0]
  n_nodes = n_users + n_items

  ego0 = jnp.concatenate([user_emb, item_emb], axis=0)
  x0 = jnp.concatenate([ego0[:, :DH], ego0[:, DH:]], axis=0)
  it = jnp.concatenate([item_emb[:, :DH], item_emb[:, DH:]], axis=0)

  per_sb = NS * CHUNK * SB_C
  ui_chunks = SB_C * (-(-adj_row.shape[0] // per_sb))
  mm_chunks = SB_C * (-(-mm_row.shape[0] // per_sb))
  arow, acol, aval = _pad_edges2(adj_row, adj_col, adj_vals, ui_chunks,
                                 n_nodes, n_nodes)
  mrow, mcol, mval = _pad_edges2(mm_row, mm_col, mm_vals, mm_chunks,
                                 n_items, n_items)

  spmm_ui = _spmm_sc(n_nodes, n_nodes, ui_chunks)
  spmm_mm = _spmm_sc(n_items, n_items, mm_chunks)

  h = spmm_mm(mcol, mrow, mval, it)
  y1 = spmm_ui(acol, arow, aval, x0)
  y2 = spmm_ui(acol, arow, aval, y1)

  u = _combine_u(x0, y1, y2, n_users, n_nodes)
  i = _combine_i(x0, y1, y2, h, n_users, n_items, n_nodes)
  return (u, i)
